# baseline (speedup 1.0000x reference)
import functools

import jax
import jax.numpy as jnp
from jax import lax
from jax.experimental import pallas as pl
from jax.experimental.pallas import tpu as pltpu
from jax.experimental.pallas import tpu_sc as plsc

N = 10000
E = 320000
D = 128
H = 16
C = 2
G = 64

NC = 2
NS = 16
NW = NC * NS
CHUNK = 128
NCHUNKS = E // CHUNK
KMAIN = NCHUNKS // NW
EXTRA = NCHUNKS - KMAIN * NW
NB = 6
NBUF = 2 * NB
ROWS_PT = 632
N_PAD = NS * ROWS_PT
NPK = N // 8
NPK_PAD = N_PAD // 8


def _sc_scatter_body(p_hbm, ei_hbm, out_hbm,
                     src_v, dst_v, rows_v, zbuf_v, agg_sh,
                     sem_i, sem_g, sem_s):
  c = lax.axis_index("c")
  s = lax.axis_index("s")
  w = c * NS + s
  base = w * KMAIN

  cp_src = pltpu.async_copy(ei_hbm.at[0, pl.ds(base, KMAIN)],
                            src_v.at[pl.ds(0, KMAIN)], sem_i)
  cp_dst = pltpu.async_copy(ei_hbm.at[1, pl.ds(base, KMAIN)],
                            dst_v.at[pl.ds(0, KMAIN)], sem_i)

  @pl.when(w < EXTRA)
  def _():
    pltpu.async_copy(ei_hbm.at[0, NW * KMAIN + w], src_v.at[KMAIN], sem_i)
    pltpu.async_copy(ei_hbm.at[1, NW * KMAIN + w], dst_v.at[KMAIN], sem_i)

  def zero_row(i, carry):
    zbuf_v[pl.ds(2 * i, 2), :] = jnp.zeros((2, 16), jnp.bfloat16)
    return carry
  lax.fori_loop(0, ROWS_PT // 2, zero_row, 0, unroll=8)
  pltpu.sync_copy(zbuf_v, agg_sh.at[pl.ds(s * ROWS_PT, ROWS_PT)])
  cp_src.wait()
  cp_dst.wait()

  @pl.when(w < EXTRA)
  def _():
    pltpu.make_async_copy(ei_hbm.at[0, NW * KMAIN + w], src_v.at[KMAIN],
                          sem_i).wait()
    pltpu.make_async_copy(ei_hbm.at[1, NW * KMAIN + w], dst_v.at[KMAIN],
                          sem_i).wait()
  plsc.subcore_barrier()

  def start_gather(j, b):
    pltpu.async_copy(p_hbm.at[src_v.at[j]], rows_v.at[b], sem_g.at[b])

  def wait_gather(j, b):
    pltpu.make_async_copy(p_hbm.at[src_v.at[j]], rows_v.at[b],
                          sem_g.at[b]).wait()

  def start_scatter(j, b):
    pltpu.async_copy(rows_v.at[b], agg_sh.at[dst_v.at[j]], sem_s.at[b],
                     add=True)

  def wait_scatter(j, b):
    pltpu.make_async_copy(rows_v.at[b], agg_sh.at[dst_v.at[j]],
                          sem_s.at[b]).wait()

  for b in range(NB):
    start_gather(b, b)

  def body(j, carry):
    b = j % NBUF
    wait_gather(j, b)
    start_scatter(j, b)

    @pl.when(j + NB < KMAIN)
    def _():
      b2 = (j + NB) % NBUF

      @pl.when(j + NB >= NBUF)
      def _():
        wait_scatter(j + NB - NBUF, b2)
      start_gather(j + NB, b2)
    return carry
  lax.fori_loop(0, KMAIN, body, 0)

  for d in range(NBUF):
    j = KMAIN - NBUF + d
    wait_scatter(j, j % NBUF)

  @pl.when(w < EXTRA)
  def _():
    pltpu.async_copy(p_hbm.at[src_v.at[KMAIN]], rows_v.at[0],
                     sem_g.at[0]).wait()
    pltpu.async_copy(rows_v.at[0], agg_sh.at[dst_v.at[KMAIN]],
                     sem_s.at[0], add=True).wait()
  plsc.subcore_barrier()

  pltpu.sync_copy(agg_sh.at[pl.ds(s * ROWS_PT, ROWS_PT)],
                  out_hbm.at[c, pl.ds(s * ROWS_PT, ROWS_PT)])


_sc_scatter = functools.partial(
    pl.kernel,
    out_type=jax.ShapeDtypeStruct((NC, N_PAD, H), jnp.bfloat16),
    mesh=plsc.VectorSubcoreMesh(core_axis_name="c", subcore_axis_name="s",
                                num_cores=NC, num_subcores=NS),
    compiler_params=pltpu.CompilerParams(use_tc_tiling_on_sc=False),
    scratch_types=[
        pltpu.VMEM((KMAIN + 1, CHUNK), jnp.int32),
        pltpu.VMEM((KMAIN + 1, CHUNK), jnp.int32),
        pltpu.VMEM((NBUF, CHUNK, H), jnp.bfloat16),
        pltpu.VMEM((ROWS_PT, H), jnp.bfloat16),
        pltpu.VMEM_SHARED((N_PAD, H), jnp.bfloat16),
        pltpu.SemaphoreType.DMA,
        pltpu.SemaphoreType.DMA((NBUF,)),
        pltpu.SemaphoreType.DMA((NBUF,)),
    ],
)(_sc_scatter_body)


def _proj1_body(xp_ref, ka_ref, kb_ref, p_ref, r_ref):
  xp = xp_ref[...]
  p = jnp.dot(xp, ka_ref[...], preferred_element_type=jnp.float32)
  r = jnp.dot(xp, kb_ref[...], preferred_element_type=jnp.float32)
  zpad = jnp.zeros((NPK_PAD - NPK, 128), jnp.float32)
  p_ref[...] = jnp.concatenate([p, zpad]).astype(jnp.bfloat16)
  r_ref[...] = jnp.concatenate([r, zpad])


def _proj1(xp, ka, kb):
  return pl.pallas_call(
      _proj1_body,
      out_shape=(jax.ShapeDtypeStruct((NPK_PAD, 128), jnp.bfloat16),
                 jax.ShapeDtypeStruct((NPK_PAD, 128), jnp.float32)),
  )(xp, ka, kb)


def _fuse_body(agg_ref, r_ref, b_ref, ka_ref, kb_ref, p_ref, r2_ref):
  agg = agg_ref[0].astype(jnp.float32) + agg_ref[1].astype(jnp.float32)
  tot = agg + r_ref[...] + b_ref[...]
  h = jnp.where(tot >= 0, tot, 0.01 * tot)
  p = jnp.dot(h, ka_ref[...], preferred_element_type=jnp.float32)
  p_ref[...] = p.astype(jnp.bfloat16)
  r2_ref[...] = jnp.dot(h, kb_ref[...], preferred_element_type=jnp.float32)


def _fuse(agg, r, b, ka, kb):
  return pl.pallas_call(
      _fuse_body,
      out_shape=(jax.ShapeDtypeStruct((NPK_PAD, 128), jnp.bfloat16),
                 jax.ShapeDtypeStruct((NPK_PAD, 128), jnp.float32)),
  )(agg, r, b, ka, kb)


def _final_body(agg_ref, r_ref, b_ref, batchT_ref, spool_ref, bc_ref,
                out_ref):
  agg = agg_ref[0].astype(jnp.float32) + agg_ref[1].astype(jnp.float32)
  h3 = agg + r_ref[...] + b_ref[...]
  h3r = h3[:NPK, :]
  segs = lax.broadcasted_iota(jnp.int32, (G, NPK), 0)
  cols = []
  for k in range(8):
    onehot_k = (batchT_ref[k:k + 1, :] == segs).astype(jnp.float32)
    dk = lax.dot_general(onehot_k, h3r, (((1,), (0,)), ((), ())),
                         preferred_element_type=jnp.float32)
    cols.append(dk[:, 16 * k:16 * (k + 1)])
  pooled_sel = jnp.concatenate(cols, axis=1)
  out_ref[...] = jnp.dot(pooled_sel, spool_ref[...],
                         preferred_element_type=jnp.float32) + bc_ref[...]


def _final(agg, r, b, batchT, spool, bc):
  return pl.pallas_call(
      _final_body,
      out_shape=jax.ShapeDtypeStruct((G, C), jnp.float32),
  )(agg, r, b, batchT, spool, bc)


def kernel(x, edge_index, batch, Wrel1, brel1, Wroot1, Wrel2, brel2, Wroot2,
           Wrel3, brel3, Wroot3, Wc, bc):
  ei3 = edge_index.reshape(2, NCHUNKS, CHUNK)
  xp = x.reshape(NPK, 8 * D)

  eye8 = jnp.eye(8, dtype=jnp.float32)
  k1a = jnp.kron(eye8, Wrel1.T)
  k1b = jnp.kron(eye8, Wroot1.T)
  k2a = jnp.kron(eye8, Wrel2.T)
  k2b = jnp.kron(eye8, Wroot2.T)
  k3a = jnp.kron(eye8, Wrel3.T)
  k3b = jnp.kron(eye8, Wroot3.T)
  b1t = jnp.tile(brel1, 8).reshape(1, 128)
  b2t = jnp.tile(brel2, 8).reshape(1, 128)
  b3t = jnp.tile(brel3, 8).reshape(1, 128)
  batchT = batch.reshape(NPK, 8).T
  spool = jnp.kron(jnp.ones((8, 1), jnp.float32), Wc.T)
  bc2 = bc.reshape(1, C)

  p1, r1 = _proj1(xp, k1a, k1b)
  agg1 = _sc_scatter(p1.reshape(N_PAD, H), ei3)
  p2, r2 = _fuse(agg1.reshape(NC, NPK_PAD, 128), r1, b1t, k2a, k2b)
  agg2 = _sc_scatter(p2.reshape(N_PAD, H), ei3)
  p3, r3 = _fuse(agg2.reshape(NC, NPK_PAD, 128), r2, b2t, k3a, k3b)
  agg3 = _sc_scatter(p3.reshape(N_PAD, H), ei3)
  return _final(agg3.reshape(NC, NPK_PAD, 128), r3, b3t, batchT, spool, bc2)


# speedup vs baseline: 1.1716x; 1.1716x over previous
import functools

import jax
import jax.numpy as jnp
from jax import lax
from jax.experimental import pallas as pl
from jax.experimental.pallas import tpu as pltpu
from jax.experimental.pallas import tpu_sc as plsc

N = 10000
E = 320000
D = 128
H = 16
C = 2
G = 64

NC = 2
NS = 16
NW = NC * NS
CHUNK = 128
NCHUNKS = E // CHUNK
KMAIN = NCHUNKS // NW
EXTRA = NCHUNKS - KMAIN * NW
NB = 6
NBUF = 2 * NB
ROWS_PT = 632
N_PAD = NS * ROWS_PT
NPK = N // 8
NPK_PAD = N_PAD // 8


def _sc_scatter_body(p_hbm, ei_hbm, out_hbm,
                     src_v, dst_v, rows_v, zbuf_v, agg_sh,
                     sem_i, sem_g, sem_s):
  c = lax.axis_index("c")
  s = lax.axis_index("s")
  w = c * NS + s
  base = w * KMAIN

  cp_src = pltpu.async_copy(ei_hbm.at[0, pl.ds(base, KMAIN)],
                            src_v.at[pl.ds(0, KMAIN)], sem_i)
  cp_dst = pltpu.async_copy(ei_hbm.at[1, pl.ds(base, KMAIN)],
                            dst_v.at[pl.ds(0, KMAIN)], sem_i)

  @pl.when(w < EXTRA)
  def _():
    pltpu.async_copy(ei_hbm.at[0, NW * KMAIN + w], src_v.at[KMAIN], sem_i)
    pltpu.async_copy(ei_hbm.at[1, NW * KMAIN + w], dst_v.at[KMAIN], sem_i)

  def zero_row(i, carry):
    zbuf_v[i] = jnp.zeros((16,), jnp.float32)
    return carry
  lax.fori_loop(0, ROWS_PT, zero_row, 0, unroll=8)
  pltpu.sync_copy(zbuf_v, agg_sh.at[pl.ds(s * ROWS_PT, ROWS_PT)])
  cp_src.wait()
  cp_dst.wait()

  @pl.when(w < EXTRA)
  def _():
    pltpu.make_async_copy(ei_hbm.at[0, NW * KMAIN + w], src_v.at[KMAIN],
                          sem_i).wait()
    pltpu.make_async_copy(ei_hbm.at[1, NW * KMAIN + w], dst_v.at[KMAIN],
                          sem_i).wait()
  plsc.subcore_barrier()

  def start_gather(j, b):
    pltpu.async_copy(p_hbm.at[src_v.at[j]], rows_v.at[b], sem_g.at[b])

  def wait_gather(j, b):
    pltpu.make_async_copy(p_hbm.at[src_v.at[j]], rows_v.at[b],
                          sem_g.at[b]).wait()

  def start_scatter(j, b):
    pltpu.async_copy(rows_v.at[b], agg_sh.at[dst_v.at[j]], sem_s.at[b],
                     add=True)

  def wait_scatter(j, b):
    pltpu.make_async_copy(rows_v.at[b], agg_sh.at[dst_v.at[j]],
                          sem_s.at[b]).wait()

  for b in range(NB):
    start_gather(b, b)

  def body(j, carry):
    b = j % NBUF
    wait_gather(j, b)
    start_scatter(j, b)

    @pl.when(j + NB < KMAIN)
    def _():
      b2 = (j + NB) % NBUF

      @pl.when(j + NB >= NBUF)
      def _():
        wait_scatter(j + NB - NBUF, b2)
      start_gather(j + NB, b2)
    return carry
  lax.fori_loop(0, KMAIN, body, 0)

  for d in range(NBUF):
    j = KMAIN - NBUF + d
    wait_scatter(j, j % NBUF)

  @pl.when(w < EXTRA)
  def _():
    pltpu.async_copy(p_hbm.at[src_v.at[KMAIN]], rows_v.at[0],
                     sem_g.at[0]).wait()
    pltpu.async_copy(rows_v.at[0], agg_sh.at[dst_v.at[KMAIN]],
                     sem_s.at[0], add=True).wait()
  plsc.subcore_barrier()

  pltpu.sync_copy(agg_sh.at[pl.ds(s * ROWS_PT, ROWS_PT)],
                  out_hbm.at[c, pl.ds(s * ROWS_PT, ROWS_PT)])


_sc_scatter = functools.partial(
    pl.kernel,
    out_type=jax.ShapeDtypeStruct((NC, N_PAD, H), jnp.float32),
    mesh=plsc.VectorSubcoreMesh(core_axis_name="c", subcore_axis_name="s",
                                num_cores=NC, num_subcores=NS),
    compiler_params=pltpu.CompilerParams(use_tc_tiling_on_sc=False),
    scratch_types=[
        pltpu.VMEM((KMAIN + 1, CHUNK), jnp.int32),
        pltpu.VMEM((KMAIN + 1, CHUNK), jnp.int32),
        pltpu.VMEM((NBUF, CHUNK, H), jnp.float32),
        pltpu.VMEM((ROWS_PT, H), jnp.float32),
        pltpu.VMEM_SHARED((N_PAD, H), jnp.float32),
        pltpu.SemaphoreType.DMA,
        pltpu.SemaphoreType.DMA((NBUF,)),
        pltpu.SemaphoreType.DMA((NBUF,)),
    ],
)(_sc_scatter_body)


_HI = lax.Precision.HIGHEST


def _proj1_body(xp_ref, ka_ref, kb_ref, p_ref, r_ref):
  xp = xp_ref[...]
  p = jnp.dot(xp, ka_ref[...], preferred_element_type=jnp.float32)
  r = jnp.dot(xp, kb_ref[...], preferred_element_type=jnp.float32)
  zpad = jnp.zeros((NPK_PAD - NPK, 128), jnp.float32)
  p_ref[...] = jnp.concatenate([p, zpad])
  r_ref[...] = jnp.concatenate([r, zpad])


def _proj1(xp, ka, kb):
  return pl.pallas_call(
      _proj1_body,
      out_shape=(jax.ShapeDtypeStruct((NPK_PAD, 128), jnp.float32),
                 jax.ShapeDtypeStruct((NPK_PAD, 128), jnp.float32)),
  )(xp, ka, kb)


def _fuse_body(agg_ref, r_ref, b_ref, ka_ref, kb_ref, p_ref, r2_ref):
  tot = agg_ref[0] + agg_ref[1] + r_ref[...] + b_ref[...]
  h = jnp.where(tot >= 0, tot, 0.01 * tot)
  p_ref[...] = jnp.dot(h, ka_ref[...], preferred_element_type=jnp.float32)
  r2_ref[...] = jnp.dot(h, kb_ref[...], preferred_element_type=jnp.float32)


def _fuse(agg, r, b, ka, kb):
  return pl.pallas_call(
      _fuse_body,
      out_shape=(jax.ShapeDtypeStruct((NPK_PAD, 128), jnp.float32),
                 jax.ShapeDtypeStruct((NPK_PAD, 128), jnp.float32)),
  )(agg, r, b, ka, kb)


def _final_body(agg_ref, r_ref, b_ref, batchT_ref, wcT_ref, bc_ref,
                out_ref):
  h3 = agg_ref[0] + agg_ref[1] + r_ref[...] + b_ref[...]
  h3r = h3[:NPK, :]
  segs = lax.broadcasted_iota(jnp.int32, (G, NPK), 0)
  pooled = jnp.zeros((G, H), jnp.float32)
  for k in range(8):
    onehot_k = (batchT_ref[k:k + 1, :] == segs).astype(jnp.float32)
    dk = lax.dot_general(onehot_k, h3r, (((1,), (0,)), ((), ())),
                         preferred_element_type=jnp.float32, precision=_HI)
    pooled = pooled + dk[:, 16 * k:16 * (k + 1)]
  out_ref[...] = jnp.dot(pooled, wcT_ref[...],
                         preferred_element_type=jnp.float32) + bc_ref[...]


def _final(agg, r, b, batchT, wcT, bc):
  return pl.pallas_call(
      _final_body,
      out_shape=jax.ShapeDtypeStruct((G, C), jnp.float32),
  )(agg, r, b, batchT, wcT, bc)


def kernel(x, edge_index, batch, Wrel1, brel1, Wroot1, Wrel2, brel2, Wroot2,
           Wrel3, brel3, Wroot3, Wc, bc):
  ei3 = edge_index.reshape(2, NCHUNKS, CHUNK)
  xp = x.reshape(NPK, 8 * D)

  eye8 = jnp.eye(8, dtype=jnp.float32)
  k1a = jnp.kron(eye8, Wrel1.T)
  k1b = jnp.kron(eye8, Wroot1.T)
  k2a = jnp.kron(eye8, Wrel2.T)
  k2b = jnp.kron(eye8, Wroot2.T)
  k3a = jnp.kron(eye8, Wrel3.T)
  k3b = jnp.kron(eye8, Wroot3.T)
  b1t = jnp.tile(brel1, 8).reshape(1, 128)
  b2t = jnp.tile(brel2, 8).reshape(1, 128)
  b3t = jnp.tile(brel3, 8).reshape(1, 128)
  batchT = batch.reshape(NPK, 8).T
  bc2 = bc.reshape(1, C)

  p1, r1 = _proj1(xp, k1a, k1b)
  agg1 = _sc_scatter(p1.reshape(N_PAD, H), ei3)
  p2, r2 = _fuse(agg1.reshape(NC, NPK_PAD, 128), r1, b1t, k2a, k2b)
  agg2 = _sc_scatter(p2.reshape(N_PAD, H), ei3)
  p3, r3 = _fuse(agg2.reshape(NC, NPK_PAD, 128), r2, b2t, k3a, k3b)
  agg3 = _sc_scatter(p3.reshape(N_PAD, H), ei3)
  return _final(agg3.reshape(NC, NPK_PAD, 128), r3, b3t, batchT, Wc.T, bc2)
